# baseline (speedup 1.0000x reference)
import functools

import jax
import jax.numpy as jnp
from jax import lax
from jax.experimental import pallas as pl
from jax.experimental.pallas import tpu as pltpu
from jax.experimental.pallas import tpu_sc as plsc

_N = 32768 * 1024
_NC = 2
_NS = 16
_NW = _NC * _NS
_PER_W = _N // _NW
_CHUNK = 32768
_NCHUNK = _PER_W // _CHUNK
_LANES = 16


def _sc_body(x_hbm, y_hbm, out_hbm, buf, accv, sem):
    c = lax.axis_index("c")
    s = lax.axis_index("s")
    wid = s * _NC + c
    base = wid * _PER_W

    for oidx, arr in enumerate((x_hbm, y_hbm)):
        def chunk_body(k, acc, arr=arr):
            pltpu.async_copy(arr.at[pl.ds(base + k * _CHUNK, _CHUNK)], buf, sem).wait()

            ones = jnp.ones((_LANES,), jnp.int32)
            zeros = jnp.zeros((_LANES,), jnp.int32)

            def inner(i, acc):
                v = buf[pl.ds(i * _LANES, _LANES)]
                return acc + jnp.where(v > 0.0, ones, zeros)

            return lax.fori_loop(0, _CHUNK // _LANES, inner, acc)

        acc = lax.fori_loop(0, _NCHUNK, chunk_body, jnp.zeros((_LANES,), jnp.int32))
        accv[...] = acc
        pltpu.sync_copy(accv, out_hbm.at[wid, oidx])


_sc_count = functools.partial(
    pl.kernel,
    out_type=jax.ShapeDtypeStruct((_NW, 2, _LANES), jnp.int32),
    mesh=plsc.VectorSubcoreMesh(core_axis_name="c", subcore_axis_name="s"),
    scratch_types=[
        pltpu.VMEM((_CHUNK,), jnp.float32),
        pltpu.VMEM((_LANES,), jnp.int32),
        pltpu.SemaphoreType.DMA,
    ],
)(_sc_body)


def kernel(x, y):
    parts = _sc_count(x.reshape(-1), y.reshape(-1))
    totals = parts.sum(axis=(0, 2))
    return jnp.maximum(totals[0], totals[1])
